# baseline (speedup 1.0000x reference)
import functools

import jax
import jax.numpy as jnp
from jax import lax
from jax.experimental import pallas as pl
from jax.experimental.pallas import tpu as pltpu
from jax.experimental.pallas import tpu_sc as plsc

N = 10000
D = 128
NSC = 6
NREL = 14
NWORK = 32
CH = 128
NCHUNK = 32
ET_PAD = NWORK * NCHUNK * CH
PER_W = ET_PAD // NWORK
NSLOT = 2
ACC_ROWS = 10240
RPT = ACC_ROWS // 16
OPT = 640
BN = 1000


def _relmm(feat, w_all):
    def body(f_ref, w_ref, o_ref):
        f = f_ref[...]
        for r in range(NREL):
            o_ref[r] = lax.dot_general(
                f, w_ref[r], (((1,), (1,)), ((), ())),
                preferred_element_type=jnp.float32)
    return pl.pallas_call(
        body,
        grid=(N // BN,),
        in_specs=[
            pl.BlockSpec((BN, D), lambda n: (n, 0)),
            pl.BlockSpec((NREL, D, D), lambda n: (0, 0, 0)),
        ],
        out_specs=pl.BlockSpec((NREL, BN, D), lambda n: (0, n, 0)),
        out_shape=jax.ShapeDtypeStruct((NREL, N, D), jnp.float32),
    )(feat, w_all)


def _sc_edge_scatter(x_flat, v_idx, u_idx):
    mesh = plsc.VectorSubcoreMesh(core_axis_name="c", subcore_axis_name="s")

    @functools.partial(
        pl.kernel,
        out_type=jax.ShapeDtypeStruct((2 * N, D), jnp.float32),
        mesh=mesh,
        scratch_types=[
            pltpu.VMEM((NCHUNK, CH), jnp.int32),
            pltpu.VMEM((NCHUNK, CH), jnp.int32),
            [pltpu.VMEM((CH, D), jnp.float32) for _ in range(NSLOT)],
            pltpu.VMEM_SHARED((ACC_ROWS, D), jnp.float32),
            [pltpu.SemaphoreType.DMA for _ in range(NSLOT)],
            [pltpu.SemaphoreType.DMA for _ in range(NSLOT)],
        ],
    )
    def k(x_hbm, v_hbm, u_hbm, out_hbm, vi_all, ui_all, rows, accum,
          gsem, ssem):
        cid = lax.axis_index("c")
        sid = lax.axis_index("s")
        wid = cid * 16 + sid

        zeros16 = jnp.zeros((16,), jnp.float32)

        @pl.loop(0, CH)
        def _(i):
            for kk in range(D // 16):
                rows[0][i, pl.ds(kk * 16, 16)] = zeros16

        base_r = pl.multiple_of(sid * RPT, 16)
        crow = pl.multiple_of(wid * NCHUNK, 8)
        zcopies = []
        off = 0
        for nrow in (CH, CH, CH, CH, RPT - 4 * CH):
            zcopies.append(pltpu.make_async_copy(
                rows[0].at[pl.ds(0, nrow)],
                accum.at[pl.ds(base_r + off, nrow)], ssem[0]))
            off += nrow
        zcopies.append(pltpu.make_async_copy(
            v_hbm.at[pl.ds(crow, NCHUNK)], vi_all, gsem[0]))
        zcopies.append(pltpu.make_async_copy(
            u_hbm.at[pl.ds(crow, NCHUNK)], ui_all, gsem[1]))
        for c in zcopies:
            c.start()
        for c in zcopies:
            c.wait()
        plsc.subcore_barrier()

        def gather(j, s):
            pltpu.async_copy(x_hbm.at[vi_all.at[j]], rows[s], gsem[s])

        def wait_gather(j, s):
            pltpu.make_async_copy(x_hbm.at[vi_all.at[j]], rows[s],
                                  gsem[s]).wait()

        def scatter(j, s):
            pltpu.async_copy(rows[s], accum.at[ui_all.at[j]], ssem[s],
                             add=True)

        def wait_scatter(j, s):
            pltpu.make_async_copy(rows[s], accum.at[ui_all.at[j]],
                                  ssem[s]).wait()

        for s in range(NSLOT):
            gather(s, s)

        @pl.loop(0, NCHUNK // NSLOT - 1)
        def _(h):
            j0 = NSLOT * h
            for s in range(NSLOT):
                wait_gather(j0 + s, s)
                scatter(j0 + s, s)
            for s in range(NSLOT):
                wait_scatter(j0 + s, s)
                gather(j0 + NSLOT + s, s)

        jlast = NCHUNK - NSLOT
        for s in range(NSLOT):
            wait_gather(jlast + s, s)
            scatter(jlast + s, s)
        for s in range(NSLOT):
            wait_scatter(jlast + s, s)
        plsc.subcore_barrier()
        ob = pl.multiple_of(sid * OPT, 16)
        obase = pl.multiple_of(cid * N, 16)

        @pl.when(sid < 15)
        def _():
            pltpu.sync_copy(accum.at[pl.ds(ob, OPT)],
                            out_hbm.at[pl.ds(obase + ob, OPT)])

        @pl.when(sid == 15)
        def _():
            pltpu.sync_copy(accum.at[pl.ds(15 * OPT, N - 15 * OPT)],
                            out_hbm.at[pl.ds(obase + 15 * OPT, N - 15 * OPT)])

    return k(x_flat, v_idx, u_idx)


def _block_tail(feat, t01, w_ctr, w_ctr2, g1, b1, g2, b2, w_next=None):
    def body(f_ref, t0_ref, t1_ref, wc_ref, wc2_ref,
             g1_ref, b1_ref, g2_ref, b2_ref, *rest):
        if len(rest) == 1:
            wn_ref, (o_ref,) = None, rest
        else:
            wn_ref, o_ref, x_ref = rest
        f = f_ref[...]
        temp = lax.dot_general(
            f, wc_ref[...], (((1,), (1,)), ((), ())),
            preferred_element_type=jnp.float32)
        temp = temp + t0_ref[...] + t1_ref[...]
        m = jnp.mean(temp, axis=-1, keepdims=True)
        v = jnp.mean(jnp.square(temp - m), axis=-1, keepdims=True)
        h = (temp - m) * lax.rsqrt(v + 1e-5) * g1_ref[...] + b1_ref[...]
        h = jnp.maximum(h, 0.0)
        h2 = lax.dot_general(
            h, wc2_ref[...], (((1,), (1,)), ((), ())),
            preferred_element_type=jnp.float32)
        m2 = jnp.mean(h2, axis=-1, keepdims=True)
        v2 = jnp.mean(jnp.square(h2 - m2), axis=-1, keepdims=True)
        n2 = (h2 - m2) * lax.rsqrt(v2 + 1e-5) * g2_ref[...] + b2_ref[...]
        newf = jnp.maximum(n2 + f, 0.0)
        o_ref[...] = newf
        if wn_ref is not None:
            for r in range(NREL):
                x_ref[r] = lax.dot_general(
                    newf, wn_ref[r], (((1,), (1,)), ((), ())),
                    preferred_element_type=jnp.float32)

    nb = N // BN
    row_spec = pl.BlockSpec((BN, D), lambda n: (n, 0))
    full_mat = pl.BlockSpec((D, D), lambda n: (0, 0))
    full_vec = pl.BlockSpec((1, D), lambda n: (0, 0))
    in_specs = [
        row_spec,
        pl.BlockSpec((BN, D), lambda n: (n, 0)),
        pl.BlockSpec((BN, D), lambda n: (n + nb, 0)),
        full_mat, full_mat,
        full_vec, full_vec, full_vec, full_vec,
    ]
    args = [feat, t01, t01, w_ctr, w_ctr2, g1, b1, g2, b2]
    out_specs = row_spec
    out_shape = jax.ShapeDtypeStruct((N, D), jnp.float32)
    if w_next is not None:
        in_specs.append(pl.BlockSpec((NREL, D, D), lambda n: (0, 0, 0)))
        args.append(w_next)
        out_specs = (row_spec,
                     pl.BlockSpec((NREL, BN, D), lambda n: (0, n, 0)))
        out_shape = (out_shape,
                     jax.ShapeDtypeStruct((NREL, N, D), jnp.float32))
    return pl.pallas_call(
        body,
        grid=(nb,),
        in_specs=in_specs,
        out_specs=out_specs,
        out_shape=out_shape,
    )(*args)


def kernel(feat, W_ctr, W_pre, W_suc, W_left, W_right, W_ctr2,
           gn1_gamma, gn1_beta, gn2_gamma, gn2_beta,
           pre_u, pre_v, suc_u, suc_v, left_u, left_v, right_u, right_v):
    offs = (jnp.arange(NSC, dtype=jnp.int32) * N)[:, None]
    v_idx = jnp.concatenate([
        (pre_v.astype(jnp.int32) + offs).reshape(-1),
        (suc_v.astype(jnp.int32) + offs + NSC * N).reshape(-1),
        left_v.astype(jnp.int32) + 2 * NSC * N,
        right_v.astype(jnp.int32) + (2 * NSC + 1) * N,
    ])
    u_idx = jnp.concatenate([
        pre_u.reshape(-1), suc_u.reshape(-1), left_u, right_u,
    ]).astype(jnp.int32)
    npad = ET_PAD - v_idx.shape[0]
    pad_ar = jnp.arange(npad, dtype=jnp.int32)
    v_idx = jnp.concatenate([v_idx, (pad_ar * 997) % (NREL * N)])
    u_idx = jnp.concatenate([u_idx, N + pad_ar % (ACC_ROWS - N)])
    v_idx = v_idx.reshape(NCHUNK, NWORK, CH).swapaxes(0, 1).reshape(-1, CH)
    u_idx = u_idx.reshape(NCHUNK, NWORK, CH).swapaxes(0, 1).reshape(-1, CH)

    w_all4 = jnp.concatenate(
        [W_pre, W_suc, W_left[:, None], W_right[:, None]], axis=1)

    x = feat
    X = _relmm(x, w_all4[0])
    for i in range(4):
        t01 = _sc_edge_scatter(X.reshape(NREL * N, D), v_idx, u_idx)
        res = _block_tail(x, t01, W_ctr[i], W_ctr2[i],
                          gn1_gamma[i][None], gn1_beta[i][None],
                          gn2_gamma[i][None], gn2_beta[i][None],
                          w_next=w_all4[i + 1] if i < 3 else None)
        if i < 3:
            x, X = res
        else:
            x = res
    return x


# speedup vs baseline: 1.0325x; 1.0325x over previous
import functools

import jax
import jax.numpy as jnp
from jax import lax
from jax.experimental import pallas as pl
from jax.experimental.pallas import tpu as pltpu
from jax.experimental.pallas import tpu_sc as plsc

N = 10000
D = 128
NSC = 6
NREL = 14
NWORK = 32
CH = 128
NCHUNK = 32
ET_PAD = NWORK * NCHUNK * CH
PER_W = ET_PAD // NWORK
NSLOT = 2
ACC_ROWS = 10240
RPT = ACC_ROWS // 16
OPT = 640
BN = 2000


def _relmm(feat, w_all):
    def body(f_ref, w_ref, o_ref):
        f = f_ref[...]
        for r in range(NREL):
            o_ref[r] = lax.dot_general(
                f, w_ref[r], (((1,), (1,)), ((), ())),
                preferred_element_type=jnp.float32)
    return pl.pallas_call(
        body,
        grid=(N // BN,),
        in_specs=[
            pl.BlockSpec((BN, D), lambda n: (n, 0)),
            pl.BlockSpec((NREL, D, D), lambda n: (0, 0, 0)),
        ],
        out_specs=pl.BlockSpec((NREL, BN, D), lambda n: (0, n, 0)),
        out_shape=jax.ShapeDtypeStruct((NREL, N, D), jnp.float32),
    )(feat, w_all)


def _sc_edge_scatter(x_flat, v_idx, u_idx):
    mesh = plsc.VectorSubcoreMesh(core_axis_name="c", subcore_axis_name="s")

    @functools.partial(
        pl.kernel,
        out_type=jax.ShapeDtypeStruct((2 * N, D), jnp.float32),
        mesh=mesh,
        scratch_types=[
            pltpu.VMEM((NCHUNK, CH), jnp.int32),
            pltpu.VMEM((NCHUNK, CH), jnp.int32),
            [pltpu.VMEM((CH, D), jnp.float32) for _ in range(NSLOT)],
            pltpu.VMEM_SHARED((ACC_ROWS, D), jnp.float32),
            [pltpu.SemaphoreType.DMA for _ in range(NSLOT)],
            [pltpu.SemaphoreType.DMA for _ in range(NSLOT)],
        ],
    )
    def k(x_hbm, v_hbm, u_hbm, out_hbm, vi_all, ui_all, rows, accum,
          gsem, ssem):
        cid = lax.axis_index("c")
        sid = lax.axis_index("s")
        wid = cid * 16 + sid

        zeros16 = jnp.zeros((16,), jnp.float32)

        @pl.loop(0, CH)
        def _(i):
            for kk in range(D // 16):
                rows[0][i, pl.ds(kk * 16, 16)] = zeros16

        base_r = pl.multiple_of(sid * RPT, 16)
        crow = pl.multiple_of(wid * NCHUNK, 8)
        zcopies = []
        off = 0
        for nrow in (CH, CH, CH, CH, RPT - 4 * CH):
            zcopies.append(pltpu.make_async_copy(
                rows[0].at[pl.ds(0, nrow)],
                accum.at[pl.ds(base_r + off, nrow)], ssem[0]))
            off += nrow
        zcopies.append(pltpu.make_async_copy(
            v_hbm.at[pl.ds(crow, NCHUNK)], vi_all, gsem[0]))
        zcopies.append(pltpu.make_async_copy(
            u_hbm.at[pl.ds(crow, NCHUNK)], ui_all, gsem[1]))
        for c in zcopies:
            c.start()
        for c in zcopies:
            c.wait()
        plsc.subcore_barrier()

        def gather(j, s):
            pltpu.async_copy(x_hbm.at[vi_all.at[j]], rows[s], gsem[s])

        def wait_gather(j, s):
            pltpu.make_async_copy(x_hbm.at[vi_all.at[j]], rows[s],
                                  gsem[s]).wait()

        def scatter(j, s):
            pltpu.async_copy(rows[s], accum.at[ui_all.at[j]], ssem[s],
                             add=True)

        def wait_scatter(j, s):
            pltpu.make_async_copy(rows[s], accum.at[ui_all.at[j]],
                                  ssem[s]).wait()

        for s in range(NSLOT):
            gather(s, s)

        @pl.loop(0, NCHUNK // NSLOT - 1)
        def _(h):
            j0 = NSLOT * h
            for s in range(NSLOT):
                wait_gather(j0 + s, s)
                scatter(j0 + s, s)
            for s in range(NSLOT):
                wait_scatter(j0 + s, s)
                gather(j0 + NSLOT + s, s)

        jlast = NCHUNK - NSLOT
        for s in range(NSLOT):
            wait_gather(jlast + s, s)
            scatter(jlast + s, s)
        for s in range(NSLOT):
            wait_scatter(jlast + s, s)
        plsc.subcore_barrier()
        ob = pl.multiple_of(sid * OPT, 16)
        obase = pl.multiple_of(cid * N, 16)

        @pl.when(sid < 15)
        def _():
            pltpu.sync_copy(accum.at[pl.ds(ob, OPT)],
                            out_hbm.at[pl.ds(obase + ob, OPT)])

        @pl.when(sid == 15)
        def _():
            pltpu.sync_copy(accum.at[pl.ds(15 * OPT, N - 15 * OPT)],
                            out_hbm.at[pl.ds(obase + 15 * OPT, N - 15 * OPT)])

    return k(x_flat, v_idx, u_idx)


def _block_tail(feat, t01, w_ctr, w_ctr2, g1, b1, g2, b2, w_next=None):
    def body(f_ref, t0_ref, t1_ref, wc_ref, wc2_ref,
             g1_ref, b1_ref, g2_ref, b2_ref, *rest):
        if len(rest) == 1:
            wn_ref, (o_ref,) = None, rest
        else:
            wn_ref, o_ref, x_ref = rest
        f = f_ref[...]
        temp = lax.dot_general(
            f, wc_ref[...], (((1,), (1,)), ((), ())),
            preferred_element_type=jnp.float32)
        temp = temp + t0_ref[...] + t1_ref[...]
        m = jnp.mean(temp, axis=-1, keepdims=True)
        v = jnp.mean(jnp.square(temp - m), axis=-1, keepdims=True)
        h = (temp - m) * lax.rsqrt(v + 1e-5) * g1_ref[...] + b1_ref[...]
        h = jnp.maximum(h, 0.0)
        h2 = lax.dot_general(
            h, wc2_ref[...], (((1,), (1,)), ((), ())),
            preferred_element_type=jnp.float32)
        m2 = jnp.mean(h2, axis=-1, keepdims=True)
        v2 = jnp.mean(jnp.square(h2 - m2), axis=-1, keepdims=True)
        n2 = (h2 - m2) * lax.rsqrt(v2 + 1e-5) * g2_ref[...] + b2_ref[...]
        newf = jnp.maximum(n2 + f, 0.0)
        o_ref[...] = newf
        if wn_ref is not None:
            for r in range(NREL):
                x_ref[r] = lax.dot_general(
                    newf, wn_ref[r], (((1,), (1,)), ((), ())),
                    preferred_element_type=jnp.float32)

    nb = N // BN
    row_spec = pl.BlockSpec((BN, D), lambda n: (n, 0))
    full_mat = pl.BlockSpec((D, D), lambda n: (0, 0))
    full_vec = pl.BlockSpec((1, D), lambda n: (0, 0))
    in_specs = [
        row_spec,
        pl.BlockSpec((BN, D), lambda n: (n, 0)),
        pl.BlockSpec((BN, D), lambda n: (n + nb, 0)),
        full_mat, full_mat,
        full_vec, full_vec, full_vec, full_vec,
    ]
    args = [feat, t01, t01, w_ctr, w_ctr2, g1, b1, g2, b2]
    out_specs = row_spec
    out_shape = jax.ShapeDtypeStruct((N, D), jnp.float32)
    if w_next is not None:
        in_specs.append(pl.BlockSpec((NREL, D, D), lambda n: (0, 0, 0)))
        args.append(w_next)
        out_specs = (row_spec,
                     pl.BlockSpec((NREL, BN, D), lambda n: (0, n, 0)))
        out_shape = (out_shape,
                     jax.ShapeDtypeStruct((NREL, N, D), jnp.float32))
    return pl.pallas_call(
        body,
        grid=(nb,),
        in_specs=in_specs,
        out_specs=out_specs,
        out_shape=out_shape,
    )(*args)


def kernel(feat, W_ctr, W_pre, W_suc, W_left, W_right, W_ctr2,
           gn1_gamma, gn1_beta, gn2_gamma, gn2_beta,
           pre_u, pre_v, suc_u, suc_v, left_u, left_v, right_u, right_v):
    offs = (jnp.arange(NSC, dtype=jnp.int32) * N)[:, None]
    v_idx = jnp.concatenate([
        (pre_v.astype(jnp.int32) + offs).reshape(-1),
        (suc_v.astype(jnp.int32) + offs + NSC * N).reshape(-1),
        left_v.astype(jnp.int32) + 2 * NSC * N,
        right_v.astype(jnp.int32) + (2 * NSC + 1) * N,
    ])
    u_idx = jnp.concatenate([
        pre_u.reshape(-1), suc_u.reshape(-1), left_u, right_u,
    ]).astype(jnp.int32)
    npad = ET_PAD - v_idx.shape[0]
    pad_ar = jnp.arange(npad, dtype=jnp.int32)
    v_idx = jnp.concatenate([v_idx, (pad_ar * 997) % (NREL * N)])
    u_idx = jnp.concatenate([u_idx, N + pad_ar % (ACC_ROWS - N)])
    v_idx = v_idx.reshape(NCHUNK, NWORK, CH).swapaxes(0, 1).reshape(-1, CH)
    u_idx = u_idx.reshape(NCHUNK, NWORK, CH).swapaxes(0, 1).reshape(-1, CH)

    w_all4 = jnp.concatenate(
        [W_pre, W_suc, W_left[:, None], W_right[:, None]], axis=1)

    x = feat
    X = _relmm(x, w_all4[0])
    for i in range(4):
        t01 = _sc_edge_scatter(X.reshape(NREL * N, D), v_idx, u_idx)
        res = _block_tail(x, t01, W_ctr[i], W_ctr2[i],
                          gn1_gamma[i][None], gn1_beta[i][None],
                          gn2_gamma[i][None], gn2_beta[i][None],
                          w_next=w_all4[i + 1] if i < 3 else None)
        if i < 3:
            x, X = res
        else:
            x = res
    return x
